# baseline (speedup 1.0000x reference)
import jax
import jax.numpy as jnp
import numpy as np
from jax.experimental import pallas as pl
from jax.experimental.pallas import tpu as pltpu

_SPECIES = (1, 6, 7, 8)
_HIGH_CUTOFF = 6.0
_N_GAUSS = 20
_W = 2.0
_B, _N = 16, 96

_OFFSETS = np.linspace(0.0, _HIGH_CUTOFF, _N_GAUSS + 1, dtype=np.float32)[1:]
_SQRTPI = float(np.sqrt(np.pi))
_PI = float(np.pi)
_SQRT2 = float(np.sqrt(2.0))

_ANG_L = (
    ((((0, 0, 0), 1.0),)),
    (((1, 0, 0), 1.0), ((0, 1, 0), 1.0), ((0, 0, 1), 1.0)),
    (((2, 0, 0), 1.0), ((1, 1, 0), _SQRT2), ((0, 2, 0), 1.0),
     ((1, 0, 1), _SQRT2), ((0, 1, 1), _SQRT2), ((0, 0, 2), 1.0)),
)


def _fp_kernel(xc_ref, xr_ref, z_ref, cnt_ref, out_ref):
    f32 = jnp.float32
    xc = xc_ref[0]
    xr = xr_ref[0]
    z = z_ref[0]
    natom = cnt_ref[0, 0, 0]

    n = _N
    dx = xr[0:1, :] - xc[:, 0:1]
    dy = xr[1:2, :] - xc[:, 1:2]
    dz = xr[2:3, :] - xc[:, 2:3]

    d2 = jnp.maximum(dx * dx + dy * dy + dz * dz, 1e-12)
    dist = jnp.sqrt(d2)
    jj = jax.lax.broadcasted_iota(jnp.int32, (n, n), 0)
    ii = jax.lax.broadcasted_iota(jnp.int32, (n, n), 1)
    valid = (dist < _HIGH_CUTOFF) & (ii != jj) & (jj < natom)
    coeffs = valid.astype(f32)

    inv_d = 1.0 / dist
    inv_d2 = inv_d * inv_d
    cut = 0.5 * (jnp.cos(dist * (_PI / _HIGH_CUTOFF)) + 1.0)
    sigma2 = jnp.log(1.0 + _W * inv_d2)
    mu = jnp.log(dist) - 0.5 * sigma2
    rsig = jax.lax.rsqrt(sigma2)
    nh = -0.5 / sigma2

    lane_i = jax.lax.broadcasted_iota(jnp.int32, (1, n), 1)
    valid_i = (lane_i < natom).astype(f32)
    base = (cut * coeffs * rsig) * valid_i

    aq = nh * mu * mu
    bq = (-2.0) * nh * mu
    rad = []
    for g in range(_N_GAUSS):
        k = float(np.log(_OFFSETS[g]))
        c = 1.0 / (float(_OFFSETS[g]) * _SQRTPI)
        rad.append((c * base) * jnp.exp(k * (nh * k + bq) + aq))

    u2 = inv_d2 * coeffs
    u3 = u2 * inv_d
    u4 = u2 * inv_d2
    mono = {(0, 0, 0): None,
            (1, 0, 0): dx, (0, 1, 0): dy, (0, 0, 1): dz,
            (2, 0, 0): dx * dx, (1, 1, 0): dx * dy, (0, 2, 0): dy * dy,
            (1, 0, 1): dx * dz, (0, 1, 1): dy * dz, (0, 0, 2): dz * dz}
    ubyl = (u2, u3, u4)
    ang_by_l = []
    for l in range(3):
        lst = []
        for (nmk, sw) in _ANG_L[l]:
            m = mono[nmk]
            if m is None:
                lst.append(ubyl[l])
            elif sw != 1.0:
                lst.append((ubyl[l] * sw) * m)
            else:
                lst.append(ubyl[l] * m)
        ang_by_l.append(lst)

    m = [(z == s).astype(f32) for s in _SPECIES]
    zero = jnp.zeros((1, n), f32)
    two = [2.0 * q for q in m]
    rows = (m + [two[1], two[2], two[3], two[0], two[2], two[3], two[0], two[1]]
            + [zero] * 4 + m + m + m + [zero] * 4)
    m32 = jnp.concatenate(rows, axis=0)

    for l in range(3):
        angs = ang_by_l[l]
        for g in range(_N_GAUSS):
            r = rad[g]
            acc = None
            for a_arr in angs:
                p = a_arr * r
                t = jax.lax.dot(m32, p, preferred_element_type=f32)
                o = t[0:16] * t[16:32]
                acc = o if acc is None else acc + o
            out_ref[0, l * _N_GAUSS + g] = acc


def kernel(coordinates, nuclear_charges, natom_counts):
    b, n, _ = coordinates.shape
    xc = coordinates.astype(jnp.float32)
    xr = jnp.transpose(xc, (0, 2, 1))
    z = nuclear_charges.astype(jnp.int32).reshape(b, 1, n)
    cnt = natom_counts.astype(jnp.int32).reshape(b, 1, 1)

    out = pl.pallas_call(
        _fp_kernel,
        grid=(b,),
        in_specs=[
            pl.BlockSpec((1, n, 3), lambda i: (i, 0, 0)),
            pl.BlockSpec((1, 3, n), lambda i: (i, 0, 0)),
            pl.BlockSpec((1, 1, n), lambda i: (i, 0, 0)),
            pl.BlockSpec((1, 1, 1), lambda i: (i, 0, 0)),
        ],
        out_specs=pl.BlockSpec((1, 60, 16, n), lambda i: (i, 0, 0, 0)),
        out_shape=jax.ShapeDtypeStruct((b, 60, 16, n), jnp.float32),
        compiler_params=pltpu.CompilerParams(
            dimension_semantics=("parallel",)),
    )(xc, xr, z, cnt)

    mb = jnp.array([0, 1, 2, 3, 4, 8, 7, 5, 9, 6], jnp.int32)
    fp = out.reshape(b, 3, _N_GAUSS, 16, n)[:, :, :, mb, :]
    fp = jnp.transpose(fp, (0, 4, 1, 3, 2))
    return fp.reshape(b, n, 3 * 10 * _N_GAUSS)


# speedup vs baseline: 1.0097x; 1.0097x over previous
import jax
import jax.numpy as jnp
import numpy as np
from jax.experimental import pallas as pl
from jax.experimental.pallas import tpu as pltpu

_SPECIES = (1, 6, 7, 8)
_HIGH_CUTOFF = 6.0
_N_GAUSS = 20
_W = 2.0
_B, _N = 16, 96

_OFFSETS = np.linspace(0.0, _HIGH_CUTOFF, _N_GAUSS + 1, dtype=np.float32)[1:]
_SQRTPI = float(np.sqrt(np.pi))
_PI = float(np.pi)
_SQRT2 = float(np.sqrt(2.0))

_ANG_L = (
    ((((0, 0, 0), 1.0),)),
    (((1, 0, 0), 1.0), ((0, 1, 0), 1.0), ((0, 0, 1), 1.0)),
    (((2, 0, 0), 1.0), ((1, 1, 0), _SQRT2), ((0, 2, 0), 1.0),
     ((1, 0, 1), _SQRT2), ((0, 1, 1), _SQRT2), ((0, 0, 2), 1.0)),
)


def _fp_kernel(xc_ref, xr_ref, z_ref, cnt_ref, out_ref):
    f32 = jnp.float32
    xc = xc_ref[0]
    xr = xr_ref[0]
    z = z_ref[0]
    natom = cnt_ref[0, 0, 0]

    n = _N
    dx = xr[0:1, :] - xc[:, 0:1]
    dy = xr[1:2, :] - xc[:, 1:2]
    dz = xr[2:3, :] - xc[:, 2:3]

    d2 = jnp.maximum(dx * dx + dy * dy + dz * dz, 1e-12)
    dist = jnp.sqrt(d2)
    jj = jax.lax.broadcasted_iota(jnp.int32, (n, n), 0)
    ii = jax.lax.broadcasted_iota(jnp.int32, (n, n), 1)
    valid = (dist < _HIGH_CUTOFF) & (ii != jj) & (jj < natom)
    coeffs = valid.astype(f32)

    inv_d = 1.0 / dist
    inv_d2 = inv_d * inv_d
    cut = 0.5 * (jnp.cos(dist * (_PI / _HIGH_CUTOFF)) + 1.0)
    sigma2 = jnp.log(1.0 + _W * inv_d2)
    mu = jnp.log(dist) - 0.5 * sigma2
    rsig = jax.lax.rsqrt(sigma2)
    nh = -0.5 / sigma2

    lane_i = jax.lax.broadcasted_iota(jnp.int32, (1, n), 1)
    valid_i = (lane_i < natom).astype(f32)
    base = (cut * coeffs * rsig) * valid_i

    rad = []
    for g in range(_N_GAUSS):
        k = float(np.log(_OFFSETS[g]))
        c = 1.0 / (float(_OFFSETS[g]) * _SQRTPI)
        cen = k - mu
        rad.append((c * base) * jnp.exp((cen * cen) * nh))

    u2 = inv_d2 * coeffs
    u3 = u2 * inv_d
    u4 = u2 * inv_d2
    mono = {(0, 0, 0): None,
            (1, 0, 0): dx, (0, 1, 0): dy, (0, 0, 1): dz,
            (2, 0, 0): dx * dx, (1, 1, 0): dx * dy, (0, 2, 0): dy * dy,
            (1, 0, 1): dx * dz, (0, 1, 1): dy * dz, (0, 0, 2): dz * dz}
    ubyl = (u2, u3, u4)
    ang_by_l = []
    for l in range(3):
        lst = []
        for (nmk, sw) in _ANG_L[l]:
            m = mono[nmk]
            if m is None:
                lst.append(ubyl[l])
            elif sw != 1.0:
                lst.append((ubyl[l] * sw) * m)
            else:
                lst.append(ubyl[l] * m)
        ang_by_l.append(lst)

    m4 = jnp.concatenate(
        [(z == s).astype(f32) for s in _SPECIES], axis=0)

    for l in range(3):
        angs = ang_by_l[l]
        for g in range(_N_GAUSS):
            r = rad[g]
            a_sq = a_c1 = a_c2 = None
            for a_arr in angs:
                p = a_arr * r
                t = jax.lax.dot(m4, p, preferred_element_type=f32)
                t1 = jnp.roll(t, -1, axis=0)
                t2 = jnp.roll(t, -2, axis=0)
                if a_sq is None:
                    a_sq, a_c1, a_c2 = t * t, t * t1, t * t2
                else:
                    a_sq += t * t
                    a_c1 += t * t1
                    a_c2 += t * t2
            blk = jnp.concatenate([a_sq, 2.0 * a_c1, 2.0 * a_c2], axis=0)
            out_ref[0, l * _N_GAUSS + g, 0:12, :] = blk


def kernel(coordinates, nuclear_charges, natom_counts):
    b, n, _ = coordinates.shape
    xc = coordinates.astype(jnp.float32)
    xr = jnp.transpose(xc, (0, 2, 1))
    z = nuclear_charges.astype(jnp.int32).reshape(b, 1, n)
    cnt = natom_counts.astype(jnp.int32).reshape(b, 1, 1)

    out = pl.pallas_call(
        _fp_kernel,
        grid=(b,),
        in_specs=[
            pl.BlockSpec((1, n, 3), lambda i: (i, 0, 0)),
            pl.BlockSpec((1, 3, n), lambda i: (i, 0, 0)),
            pl.BlockSpec((1, 1, n), lambda i: (i, 0, 0)),
            pl.BlockSpec((1, 1, 1), lambda i: (i, 0, 0)),
        ],
        out_specs=pl.BlockSpec((1, 60, 16, n), lambda i: (i, 0, 0, 0)),
        out_shape=jax.ShapeDtypeStruct((b, 60, 16, n), jnp.float32),
        compiler_params=pltpu.CompilerParams(
            dimension_semantics=("parallel",)),
    )(xc, xr, z, cnt)

    mb = jnp.array([0, 1, 2, 3, 4, 8, 7, 5, 9, 6], jnp.int32)
    fp = out.reshape(b, 3, _N_GAUSS, 16, n)[:, :, :, mb, :]
    fp = jnp.transpose(fp, (0, 4, 1, 3, 2))
    return fp.reshape(b, n, 3 * 10 * _N_GAUSS)


# speedup vs baseline: 1.6012x; 1.5858x over previous
import jax
import jax.numpy as jnp
import numpy as np
from jax.experimental import pallas as pl
from jax.experimental.pallas import tpu as pltpu

_SPECIES = (1, 6, 7, 8)
_HIGH_CUTOFF = 6.0
_N_GAUSS = 20
_W = 2.0
_B, _N = 16, 96

_OFFSETS = np.linspace(0.0, _HIGH_CUTOFF, _N_GAUSS + 1, dtype=np.float32)[1:]
_SQRTPI = float(np.sqrt(np.pi))
_PI = float(np.pi)
_SQRT2 = float(np.sqrt(2.0))

_ANG_L = (
    ((((0, 0, 0), 1.0),)),
    (((1, 0, 0), 1.0), ((0, 1, 0), 1.0), ((0, 0, 1), 1.0)),
    (((2, 0, 0), 1.0), ((1, 1, 0), _SQRT2), ((0, 2, 0), 1.0),
     ((1, 0, 1), _SQRT2), ((0, 1, 1), _SQRT2), ((0, 0, 2), 1.0)),
)


def _fp_kernel(xc_ref, xr_ref, z_ref, cnt_ref, out_ref):
    f32 = jnp.float32
    xc = xc_ref[0]
    xr = xr_ref[0]
    z = z_ref[0]
    natom = cnt_ref[0, 0, 0]

    n = _N
    dx = xr[0:1, :] - xc[:, 0:1]
    dy = xr[1:2, :] - xc[:, 1:2]
    dz = xr[2:3, :] - xc[:, 2:3]

    d2 = jnp.maximum(dx * dx + dy * dy + dz * dz, 1e-12)
    dist = jnp.sqrt(d2)
    jj = jax.lax.broadcasted_iota(jnp.int32, (n, n), 0)
    ii = jax.lax.broadcasted_iota(jnp.int32, (n, n), 1)
    valid = (dist < _HIGH_CUTOFF) & (ii != jj) & (jj < natom)
    coeffs = valid.astype(f32)

    inv_d = 1.0 / dist
    inv_d2 = inv_d * inv_d
    cut = 0.5 * (jnp.cos(dist * (_PI / _HIGH_CUTOFF)) + 1.0)
    sigma2 = jnp.log(1.0 + _W * inv_d2)
    mu = jnp.log(dist) - 0.5 * sigma2
    rsig = jax.lax.rsqrt(sigma2)
    nh = -0.5 / sigma2

    lane_i = jax.lax.broadcasted_iota(jnp.int32, (1, n), 1)
    valid_i = (lane_i < natom).astype(f32)
    base = (cut * coeffs * rsig) * valid_i

    rad = []
    for g in range(_N_GAUSS):
        k = float(np.log(_OFFSETS[g]))
        c = 1.0 / (float(_OFFSETS[g]) * _SQRTPI)
        cen = k - mu
        rad.append((c * base) * jnp.exp((cen * cen) * nh))

    u2 = inv_d2 * coeffs
    u3 = u2 * inv_d
    u4 = u2 * inv_d2
    mono = {(0, 0, 0): None,
            (1, 0, 0): dx, (0, 1, 0): dy, (0, 0, 1): dz,
            (2, 0, 0): dx * dx, (1, 1, 0): dx * dy, (0, 2, 0): dy * dy,
            (1, 0, 1): dx * dz, (0, 1, 1): dy * dz, (0, 0, 2): dz * dz}
    ubyl = (u2, u3, u4)
    ang_by_l = []
    for l in range(3):
        lst = []
        for (nmk, sw) in _ANG_L[l]:
            m = mono[nmk]
            if m is None:
                lst.append(ubyl[l])
            elif sw != 1.0:
                lst.append((ubyl[l] * sw) * m)
            else:
                lst.append(ubyl[l] * m)
        ang_by_l.append(lst)

    m4 = jnp.concatenate(
        [(z == s).astype(f32) for s in _SPECIES], axis=0)

    for l in range(3):
        angs = ang_by_l[l]
        for g in range(_N_GAUSS):
            r = rad[g]
            a_sq = a_c1 = a_c2 = None
            for a_arr in angs:
                p = a_arr * r
                t = jax.lax.dot(m4, p, preferred_element_type=f32)
                t1 = jnp.roll(t, -1, axis=0)
                t2 = jnp.roll(t, -2, axis=0)
                if a_sq is None:
                    a_sq, a_c1, a_c2 = t * t, t * t1, t * t2
                else:
                    a_sq += t * t
                    a_c1 += t * t1
                    a_c2 += t * t2
            c1 = 2.0 * a_c1
            c2 = 2.0 * a_c2
            blk = jnp.concatenate(
                [a_sq, c1[0:1], c2[0:1], c1[3:4], c1[1:2], c2[1:2], c1[2:3]],
                axis=0)
            out_ref[0, l * _N_GAUSS + g] = blk


def kernel(coordinates, nuclear_charges, natom_counts):
    b, n, _ = coordinates.shape
    xc = coordinates.astype(jnp.float32)
    xr = jnp.transpose(xc, (0, 2, 1))
    z = nuclear_charges.astype(jnp.int32).reshape(b, 1, n)
    cnt = natom_counts.astype(jnp.int32).reshape(b, 1, 1)

    out = pl.pallas_call(
        _fp_kernel,
        grid=(b,),
        in_specs=[
            pl.BlockSpec((1, n, 3), lambda i: (i, 0, 0)),
            pl.BlockSpec((1, 3, n), lambda i: (i, 0, 0)),
            pl.BlockSpec((1, 1, n), lambda i: (i, 0, 0)),
            pl.BlockSpec((1, 1, 1), lambda i: (i, 0, 0)),
        ],
        out_specs=pl.BlockSpec((1, 60, 10, n), lambda i: (i, 0, 0, 0)),
        out_shape=jax.ShapeDtypeStruct((b, 60, 10, n), jnp.float32),
        compiler_params=pltpu.CompilerParams(
            dimension_semantics=("parallel",)),
    )(xc, xr, z, cnt)

    fp = out.reshape(b, 3, _N_GAUSS, 10, n)
    fp = jnp.transpose(fp, (0, 4, 1, 3, 2))
    return fp.reshape(b, n, 3 * 10 * _N_GAUSS)


# speedup vs baseline: 1.6709x; 1.0435x over previous
import jax
import jax.numpy as jnp
import numpy as np
from jax.experimental import pallas as pl
from jax.experimental.pallas import tpu as pltpu

_SPECIES = (1, 6, 7, 8)
_HIGH_CUTOFF = 6.0
_N_GAUSS = 20
_W = 2.0
_B, _N = 16, 96

_OFFSETS = np.linspace(0.0, _HIGH_CUTOFF, _N_GAUSS + 1, dtype=np.float32)[1:]
_SQRTPI = float(np.sqrt(np.pi))
_PI = float(np.pi)
_SQRT2 = float(np.sqrt(2.0))

_ANG_L = (
    ((((0, 0, 0), 1.0),)),
    (((1, 0, 0), 1.0), ((0, 1, 0), 1.0), ((0, 0, 1), 1.0)),
    (((2, 0, 0), 1.0), ((1, 1, 0), _SQRT2), ((0, 2, 0), 1.0),
     ((1, 0, 1), _SQRT2), ((0, 1, 1), _SQRT2), ((0, 0, 2), 1.0)),
)


_BPB = 2


def _fp_kernel(xc_ref, xr_ref, z_ref, cnt_ref, out_ref):
    for bi in range(_BPB):
        _fp_one(bi, xc_ref, xr_ref, z_ref, cnt_ref, out_ref)


def _fp_one(bi, xc_ref, xr_ref, z_ref, cnt_ref, out_ref):
    f32 = jnp.float32
    xc = xc_ref[bi]
    xr = xr_ref[bi]
    z = z_ref[bi]
    natom = cnt_ref[bi, 0, 0]

    n = _N
    dx = xr[0:1, :] - xc[:, 0:1]
    dy = xr[1:2, :] - xc[:, 1:2]
    dz = xr[2:3, :] - xc[:, 2:3]

    d2 = jnp.maximum(dx * dx + dy * dy + dz * dz, 1e-12)
    dist = jnp.sqrt(d2)
    jj = jax.lax.broadcasted_iota(jnp.int32, (n, n), 0)
    ii = jax.lax.broadcasted_iota(jnp.int32, (n, n), 1)
    valid = (dist < _HIGH_CUTOFF) & (ii != jj) & (jj < natom)
    coeffs = valid.astype(f32)

    inv_d = 1.0 / dist
    inv_d2 = inv_d * inv_d
    cut = 0.5 * (jnp.cos(dist * (_PI / _HIGH_CUTOFF)) + 1.0)
    sigma2 = jnp.log(1.0 + _W * inv_d2)
    mu = jnp.log(dist) - 0.5 * sigma2
    rsig = jax.lax.rsqrt(sigma2)
    nh = -0.5 / sigma2

    lane_i = jax.lax.broadcasted_iota(jnp.int32, (1, n), 1)
    valid_i = (lane_i < natom).astype(f32)
    base = (cut * coeffs * rsig) * valid_i

    rad = []
    for g in range(_N_GAUSS):
        k = float(np.log(_OFFSETS[g]))
        c = 1.0 / (float(_OFFSETS[g]) * _SQRTPI)
        cen = k - mu
        rad.append((c * base) * jnp.exp((cen * cen) * nh))

    u2 = inv_d2 * coeffs
    u3 = u2 * inv_d
    u4 = u2 * inv_d2
    mono = {(0, 0, 0): None,
            (1, 0, 0): dx, (0, 1, 0): dy, (0, 0, 1): dz,
            (2, 0, 0): dx * dx, (1, 1, 0): dx * dy, (0, 2, 0): dy * dy,
            (1, 0, 1): dx * dz, (0, 1, 1): dy * dz, (0, 0, 2): dz * dz}
    ubyl = (u2, u3, u4)
    ang_by_l = []
    for l in range(3):
        lst = []
        for (nmk, sw) in _ANG_L[l]:
            m = mono[nmk]
            if m is None:
                lst.append(ubyl[l])
            elif sw != 1.0:
                lst.append((ubyl[l] * sw) * m)
            else:
                lst.append(ubyl[l] * m)
        ang_by_l.append(lst)

    m4 = jnp.concatenate(
        [(z == s).astype(f32) for s in _SPECIES], axis=0)

    for l in range(3):
        angs = ang_by_l[l]
        for g in range(_N_GAUSS):
            r = rad[g]
            a_sq = a_c1 = a_c2 = None
            for a_arr in angs:
                p = a_arr * r
                t = jax.lax.dot(m4, p, preferred_element_type=f32)
                t1 = jnp.roll(t, -1, axis=0)
                t2 = jnp.roll(t, -2, axis=0)
                if a_sq is None:
                    a_sq, a_c1, a_c2 = t * t, t * t1, t * t2
                else:
                    a_sq += t * t
                    a_c1 += t * t1
                    a_c2 += t * t2
            c1 = 2.0 * a_c1
            c2 = 2.0 * a_c2
            blk = jnp.concatenate(
                [a_sq, c1[0:1], c2[0:1], c1[3:4], c1[1:2], c2[1:2], c1[2:3]],
                axis=0)
            out_ref[bi, l * _N_GAUSS + g] = blk


def kernel(coordinates, nuclear_charges, natom_counts):
    b, n, _ = coordinates.shape
    xc = coordinates.astype(jnp.float32)
    xr = jnp.transpose(xc, (0, 2, 1))
    z = nuclear_charges.astype(jnp.int32).reshape(b, 1, n)
    cnt = natom_counts.astype(jnp.int32).reshape(b, 1, 1)

    out = pl.pallas_call(
        _fp_kernel,
        grid=(b // _BPB,),
        in_specs=[
            pl.BlockSpec((_BPB, n, 3), lambda i: (i, 0, 0)),
            pl.BlockSpec((_BPB, 3, n), lambda i: (i, 0, 0)),
            pl.BlockSpec((_BPB, 1, n), lambda i: (i, 0, 0)),
            pl.BlockSpec((_BPB, 1, 1), lambda i: (i, 0, 0)),
        ],
        out_specs=pl.BlockSpec((_BPB, 60, 10, n), lambda i: (i, 0, 0, 0)),
        out_shape=jax.ShapeDtypeStruct((b, 60, 10, n), jnp.float32),
        compiler_params=pltpu.CompilerParams(
            dimension_semantics=("parallel",)),
    )(xc, xr, z, cnt)

    fp = out.reshape(b, 3, _N_GAUSS, 10, n)
    fp = jnp.transpose(fp, (0, 4, 1, 3, 2))
    return fp.reshape(b, n, 3 * 10 * _N_GAUSS)
